# Initial kernel scaffold; baseline (speedup 1.0000x reference)
import jax
import jax.numpy as jnp
from jax.experimental import pallas as pl


def kernel(x, edge_index, edge_attr, batch, lin0_W, lin0_b, en1_W, en1_b, en2_W, en2_b, root, conv_b, gru_Wih, gru_Whh, gru_bih, gru_bhh, s2s_Wih, s2s_Whh, s2s_bih, s2s_bhh, mem_Wih, mem_Whh, mem_bih, mem_bhh, mlp1_W, mlp1_b, mlp2_W, mlp2_b):
    raise NotImplementedError("write your pallas kernel here")


# speedup vs baseline: 1.4080x; 1.4080x over previous
import functools

import jax
import jax.numpy as jnp
from jax import lax
from jax.experimental import pallas as pl
from jax.experimental.pallas import tpu as pltpu
from jax.experimental.pallas import tpu_sc as plsc

N = 10000
E = 160000
D = 32
G = 200
GP = 256

NC = 2
NS = 16
NW = NC * NS
CH = 128
NCH = 40
EPAD = NW * NCH * CH
NPAD = 10240
NPS = NPAD // NS


@functools.lru_cache(maxsize=None)
def _sc_gather_kernel():
    mesh = plsc.VectorSubcoreMesh(core_axis_name="c", subcore_axis_name="s")

    @functools.partial(
        pl.kernel,
        mesh=mesh,
        out_type=jax.ShapeDtypeStruct((NW, NCH, CH, D), jnp.float32),
        scratch_types=[
            pltpu.VMEM((NCH, CH), jnp.int32),
            pltpu.VMEM((CH, D), jnp.float32),
            pltpu.SemaphoreType.DMA,
        ],
        compiler_params=pltpu.CompilerParams(use_tc_tiling_on_sc=False),
    )
    def gather(table_hbm, idx_hbm, out_hbm, idx_v, rows_v, sem):
        wid = lax.axis_index("s") * NC + lax.axis_index("c")
        pltpu.sync_copy(idx_hbm.at[wid], idx_v)

        def body(j, carry):
            pltpu.async_copy(table_hbm.at[idx_v.at[j]], rows_v, sem).wait()
            pltpu.sync_copy(rows_v, out_hbm.at[wid, j])
            return carry

        lax.fori_loop(0, NCH, body, 0)

    return gather


def _sc_gather(table, idx_r):
    return _sc_gather_kernel()(table, idx_r)


@functools.lru_cache(maxsize=None)
def _sc_scatter_kernel():
    mesh = plsc.VectorSubcoreMesh(core_axis_name="c", subcore_axis_name="s")

    @functools.partial(
        pl.kernel,
        mesh=mesh,
        out_type=jax.ShapeDtypeStruct((NC, NPAD, D), jnp.float32),
        scratch_types=[
            pltpu.VMEM((NCH, CH), jnp.int32),
            pltpu.VMEM((CH, D), jnp.float32),
            pltpu.VMEM_SHARED((NPAD, D), jnp.float32),
            pltpu.SemaphoreType.DMA,
        ],
        compiler_params=pltpu.CompilerParams(use_tc_tiling_on_sc=False),
    )
    def scatter(msg_hbm, dst_hbm, zeros_hbm, out_hbm, idx_v, rows_v,
                acc_sh, sem):
        cid = lax.axis_index("c")
        sid = lax.axis_index("s")
        wid = sid * NC + cid
        pltpu.sync_copy(zeros_hbm.at[pl.ds(sid * NPS, NPS)],
                        acc_sh.at[pl.ds(sid * NPS, NPS)])
        plsc.subcore_barrier()
        pltpu.sync_copy(dst_hbm.at[wid], idx_v)

        def body(j, carry):
            pltpu.async_copy(msg_hbm.at[wid, j], rows_v, sem).wait()
            pltpu.sync_copy(rows_v, acc_sh.at[idx_v.at[j]], add=True)
            return carry

        lax.fori_loop(0, NCH, body, 0)
        plsc.subcore_barrier()
        pltpu.sync_copy(acc_sh.at[pl.ds(sid * NPS, NPS)],
                        out_hbm.at[cid, pl.ds(sid * NPS, NPS)])

    return scatter


def _sc_scatter_add(msg_r, dst_r, zeros_npad):
    return _sc_scatter_kernel()(msg_r, dst_r, zeros_npad)


def _relu_mm_body(x_ref, w_ref, b_ref, o_ref):
    o_ref[...] = jnp.maximum(
        jnp.dot(x_ref[...], w_ref[...], preferred_element_type=jnp.float32)
        + b_ref[...], 0.0)


def _relu_mm(x, wT, b2, block_rows):
    rows, kdim = x.shape
    cols = wT.shape[1]
    grid = rows // block_rows
    return pl.pallas_call(
        _relu_mm_body,
        grid=(grid,),
        in_specs=[
            pl.BlockSpec((block_rows, kdim), lambda i: (i, 0)),
            pl.BlockSpec((kdim, cols), lambda i: (0, 0)),
            pl.BlockSpec((1, cols), lambda i: (0, 0)),
        ],
        out_specs=pl.BlockSpec((block_rows, cols), lambda i: (i, 0)),
        out_shape=jax.ShapeDtypeStruct((rows, cols), jnp.float32),
    )(x, wT, b2)


BM = 4096


def _msg_body(hs_ref, z_ref, m_ref, b2_ref, o_ref):
    hs = hs_ref[...]
    zb = z_ref[...]
    acc = jnp.zeros((BM, D), jnp.float32)
    for d in range(D):
        acc = acc + hs[:, d:d + 1] * (
            jnp.dot(zb, m_ref[d * D:(d + 1) * D, :],
                    preferred_element_type=jnp.float32) + b2_ref[d:d + 1, :])
    o_ref[...] = acc


def _msg(h_src, z, m, b2):
    return pl.pallas_call(
        _msg_body,
        grid=(EPAD // BM,),
        in_specs=[
            pl.BlockSpec((BM, D), lambda i: (i, 0)),
            pl.BlockSpec((BM, D), lambda i: (i, 0)),
            pl.BlockSpec((D * D, D), lambda i: (0, 0)),
            pl.BlockSpec((D, D), lambda i: (0, 0)),
        ],
        out_specs=pl.BlockSpec((BM, D), lambda i: (i, 0)),
        out_shape=jax.ShapeDtypeStruct((EPAD, D), jnp.float32),
    )(h_src, z, m, b2)


def _gru_body(a0_ref, a1_ref, h_ref, root_ref, cb_ref, wih_ref, whh_ref,
              bih_ref, bhh_ref, o_ref):
    h = h_ref[...]
    agg = a0_ref[...] + a1_ref[...]
    m = jnp.maximum(
        agg + jnp.dot(h, root_ref[...], preferred_element_type=jnp.float32)
        + cb_ref[...], 0.0)
    gx = jnp.dot(m, wih_ref[...], preferred_element_type=jnp.float32) + bih_ref[...]
    gh = jnp.dot(h, whh_ref[...], preferred_element_type=jnp.float32) + bhh_ref[...]
    r = jax.nn.sigmoid(gx[:, :D] + gh[:, :D])
    zz = jax.nn.sigmoid(gx[:, D:2 * D] + gh[:, D:2 * D])
    n = jnp.tanh(gx[:, 2 * D:] + r * gh[:, 2 * D:])
    o_ref[...] = (1.0 - zz) * n + zz * h


def _gru(a0, a1, h, root, cb, wihT, whhT, bih, bhh):
    full = lambda s: pl.BlockSpec(s, lambda: (0,) * len(s))
    return pl.pallas_call(
        _gru_body,
        in_specs=[full((N, D)), full((N, D)), full((N, D)), full((D, D)),
                  full((1, D)), full((D, 3 * D)), full((D, 3 * D)),
                  full((1, 3 * D)), full((1, 3 * D))],
        out_specs=full((N, D)),
        out_shape=jax.ShapeDtypeStruct((N, D), jnp.float32),
    )(a0, a1, h, root, cb, wihT, whhT, bih, bhh)


def _lstm_gates(g, c):
    i = jax.nn.sigmoid(g[:, :D])
    f = jax.nn.sigmoid(g[:, D:2 * D])
    gg = jnp.tanh(g[:, 2 * D:3 * D])
    o = jax.nn.sigmoid(g[:, 3 * D:])
    c_new = f * c + i * gg
    h_new = o * jnp.tanh(c_new)
    return h_new, c_new


def _s2s_body(out_ref, bcol_ref, brow_ref, s2s_wih_ref, s2s_whh_ref,
              s2s_b_ref, mem_wih_ref, mem_b_ref, mlp1_ref, mlp1b_ref,
              mlp2_ref, mlp2b_ref, v_ref, hx_ref, cx_ref):
    out = out_ref[...]
    bcol = bcol_ref[...]
    brow = brow_ref[...]
    g_row = lax.broadcasted_iota(jnp.int32, (1, GP), 1)
    g_col = lax.broadcasted_iota(jnp.int32, (GP, 1), 0)
    onehot_b = bcol == g_row
    onehot_f = onehot_b.astype(jnp.float32)
    onehotT_f = (g_col == brow).astype(jnp.float32)

    qh = jnp.zeros((GP, D), jnp.float32)
    qc = jnp.zeros((GP, D), jnp.float32)
    q_star = jnp.zeros((GP, 2 * D), jnp.float32)
    for _ in range(6):
        g = (jnp.dot(q_star, s2s_wih_ref[...],
                     preferred_element_type=jnp.float32)
             + jnp.dot(qh, s2s_whh_ref[...],
                       preferred_element_type=jnp.float32) + s2s_b_ref[...])
        qh, qc = _lstm_gates(g, qc)
        qhb = jnp.dot(onehot_f, qh, preferred_element_type=jnp.float32)
        e = jnp.sum(out * qhb, axis=1, keepdims=True)
        emax = jnp.max(jnp.where(onehot_b, e, -1e30), axis=0, keepdims=True)
        emaxb = jnp.sum(jnp.where(onehot_b, emax, 0.0), axis=1, keepdims=True)
        a = jnp.exp(e - emaxb)
        asum = jnp.sum(a * onehot_f, axis=0, keepdims=True)
        asumb = jnp.sum(jnp.where(onehot_b, asum, 0.0), axis=1, keepdims=True)
        an = a / (asumb + 1e-16)
        r = jnp.dot(onehotT_f, an * out, preferred_element_type=jnp.float32)
        q_star = jnp.concatenate([qh, r], axis=1)

    g = (jnp.dot(q_star, mem_wih_ref[...], preferred_element_type=jnp.float32)
         + mem_b_ref[...])
    hx, cx = _lstm_gates(g, jnp.zeros((GP, D), jnp.float32))
    t = jnp.maximum(
        jnp.dot(hx, mlp1_ref[...], preferred_element_type=jnp.float32)
        + mlp1b_ref[...], 0.0)
    v_ref[...] = jnp.sum(t * mlp2_ref[...], axis=1, keepdims=True) + mlp2b_ref[...]
    hx_ref[...] = hx
    cx_ref[...] = cx


def _s2s(out_h, bcol, brow, s2s_wihT, s2s_whhT, s2s_b, mem_wihT, mem_b,
         mlp1T, mlp1b, mlp2row, mlp2b):
    full = lambda s: pl.BlockSpec(s, lambda: (0,) * len(s))
    return pl.pallas_call(
        _s2s_body,
        in_specs=[full((N, D)), full((N, 1)), full((1, N)),
                  full((2 * D, 4 * D)), full((D, 4 * D)), full((1, 4 * D)),
                  full((2 * D, 4 * D)), full((1, 4 * D)),
                  full((D, D)), full((1, D)), full((1, D)), full((1, 1))],
        out_specs=[full((GP, 1)), full((GP, D)), full((GP, D))],
        out_shape=[jax.ShapeDtypeStruct((GP, 1), jnp.float32),
                   jax.ShapeDtypeStruct((GP, D), jnp.float32),
                   jax.ShapeDtypeStruct((GP, D), jnp.float32)],
    )(out_h, bcol, brow, s2s_wihT, s2s_whhT, s2s_b, mem_wihT, mem_b,
      mlp1T, mlp1b, mlp2row, mlp2b)


def kernel(x, edge_index, edge_attr, batch, lin0_W, lin0_b, en1_W, en1_b,
           en2_W, en2_b, root, conv_b, gru_Wih, gru_Whh, gru_bih, gru_bhh,
           s2s_Wih, s2s_Whh, s2s_bih, s2s_bhh, mem_Wih, mem_Whh, mem_bih,
           mem_bhh, mlp1_W, mlp1_b, mlp2_W, mlp2_b):
    f32 = jnp.float32
    src = edge_index[0]
    dst = edge_index[1]

    src_r = jnp.pad(src, (0, EPAD - E)).reshape(NW, NCH, CH)
    dst_r = jnp.pad(dst, (0, EPAD - E), constant_values=N).reshape(NW, NCH, CH)
    ea8 = jnp.pad(edge_attr, ((0, EPAD - E), (0, 4)))
    en1_WT8 = jnp.pad(en1_W.T, ((0, 4), (0, 0)))
    x8 = jnp.pad(x, ((0, 0), (0, 5)))
    lin0_WT8 = jnp.pad(lin0_W.T, ((0, 5), (0, 0)))
    m_mat = en2_W.reshape(D, D, D).transpose(0, 2, 1).reshape(D * D, D)
    b2 = en2_b.reshape(D, D)
    zeros_npad = jnp.zeros((NPAD, D), f32)
    bcol = batch.reshape(N, 1)
    brow = batch.reshape(1, N)

    z = _relu_mm(ea8, en1_WT8, en1_b.reshape(1, D), BM)
    h = _relu_mm(x8, lin0_WT8, lin0_b.reshape(1, D), N)

    gru_wihT = gru_Wih.T
    gru_whhT = gru_Whh.T
    for _ in range(6):
        h_src = _sc_gather(h, src_r).reshape(EPAD, D)
        msg = _msg(h_src, z, m_mat, b2)
        aggs = _sc_scatter_add(msg.reshape(NW, NCH, CH, D), dst_r, zeros_npad)
        h = _gru(aggs[0, :N], aggs[1, :N], h, root, conv_b.reshape(1, D),
                 gru_wihT, gru_whhT, gru_bih.reshape(1, 3 * D),
                 gru_bhh.reshape(1, 3 * D))

    v_p, hx_p, cx_p = _s2s(
        h, bcol, brow, s2s_Wih.T, s2s_Whh.T,
        (s2s_bih + s2s_bhh).reshape(1, 4 * D), mem_Wih.T,
        (mem_bih + mem_bhh).reshape(1, 4 * D), mlp1_W.T,
        mlp1_b.reshape(1, D), mlp2_W.reshape(1, D), mlp2_b.reshape(1, 1))

    return v_p[:G][None], hx_p[:G][None], cx_p[:G][None]
